# baseline (speedup 1.0000x reference)
import functools

import jax
import jax.numpy as jnp
from jax import lax
from jax.experimental import pallas as pl
from jax.experimental.pallas import tpu as pltpu
from jax.experimental.pallas import tpu_sc as plsc

_NC = 2
_NS = 16
_NW = _NC * _NS

_DIM = 128
_CTX = 7
_PAIRS = _CTX - 1

_W_CHUNK = 32
_ROWS_CHUNK = _W_CHUNK * _CTX

_LOG1P_Q = (
    0.999999518605956,
    -0.49996356582566853,
    0.332652558440823,
    -0.24453388970156367,
    0.17659864767022154,
    -0.10679931279902478,
    0.04365928884096363,
    -0.008466410403222956,
)


def _sc_loss_partials(emb, pos_idx, neg_idx, pos_b, neg_b):
    pos_w_per_tile = pos_b // _NW
    neg_w_per_tile = neg_b // _NW
    pos_scale = 1.0 / float(pos_b * _PAIRS)
    neg_scale = 1.0 / float(neg_b * _PAIRS)
    max_idx_per_tile = max(pos_w_per_tile, neg_w_per_tile) * _CTX
    mesh = plsc.VectorSubcoreMesh(core_axis_name="c", subcore_axis_name="s")

    @functools.partial(
        pl.kernel,
        mesh=mesh,
        out_type=jax.ShapeDtypeStruct((_NW * 8, _DIM), jnp.float32),
        scratch_types=[
            pltpu.VMEM((max_idx_per_tile,), jnp.int32),
            pltpu.VMEM((2, _ROWS_CHUNK, _DIM), jnp.float32),
            pltpu.VMEM((8, _DIM), jnp.float32),
            pltpu.SemaphoreType.DMA,
            pltpu.SemaphoreType.DMA,
        ],
    )
    def kern(emb_h, pos_h, neg_h, out_h, idx_all, rows_v, acc_v, gsem0, gsem1):
        wid = lax.axis_index("s") * _NC + lax.axis_index("c")
        gsems = [gsem0, gsem1]
        zeros = jnp.zeros((16,), jnp.float32)
        for r in range(8):
            for k in range(8):
                acc_v[r, pl.ds(k * 16, 16)] = zeros

        row_iota = lax.iota(jnp.int32, 16)
        dnums = lax.GatherDimensionNumbers(
            offset_dims=(), collapsed_slice_dims=(0,), start_index_map=(0,)
        )
        perms = [
            (row_iota ^ m)[:, None] for m in (8, 4, 2, 1)
        ]
        sel_masks = [(row_iota & bit) != 0 for bit in (1, 2, 4, 8)]

        def hsum(x):
            for perm in perms:
                x = x + lax.gather(
                    x, perm, dnums, (1,),
                    mode=lax.GatherScatterMode.PROMISE_IN_BOUNDS,
                )
            return x

        def issue_gather(ci, b, sem):
            off = pl.multiple_of(ci * _ROWS_CHUNK, 8)
            pltpu.async_copy(
                emb_h.at[idx_all.at[pl.ds(off, 128)]],
                rows_v.at[b].at[pl.ds(0, 128)],
                sem,
            )
            pltpu.async_copy(
                emb_h.at[idx_all.at[pl.ds(off + 128, 96)]],
                rows_v.at[b].at[pl.ds(128, 96)],
                sem,
            )

        def wait_gather(b, sem):
            pltpu.make_async_copy(
                emb_h.at[idx_all.at[pl.ds(0, 128)]],
                rows_v.at[b].at[pl.ds(0, 128)],
                sem,
            ).wait()
            pltpu.make_async_copy(
                emb_h.at[idx_all.at[pl.ds(0, 96)]],
                rows_v.at[b].at[pl.ds(128, 96)],
                sem,
            ).wait()

        def flush_group(group, acc_row, is_pos, scale):
            cur = group
            for mask in sel_masks:
                cur = [
                    jnp.where(mask, cur[2 * j + 1], cur[2 * j])
                    for j in range(len(cur) // 2)
                ]
            x = cur[0]
            u = jnp.exp(-jnp.abs(x))
            c = [jnp.float32(v) for v in _LOG1P_Q]
            u2 = u * u
            u4 = u2 * u2
            q = (c[0] + c[1] * u + u2 * (c[2] + c[3] * u)) + u4 * (
                (c[4] + c[5] * u) + u2 * (c[6] + c[7] * u)
            )
            l1p = u * q
            if is_pos:
                contrib = l1p - jnp.minimum(x, 0.0)
            else:
                contrib = l1p + jnp.maximum(x, 0.0)
            acc_v[acc_row, pl.ds(0, 16)] = (
                acc_v[acc_row, pl.ds(0, 16)] + contrib * jnp.float32(scale)
            )

        def run_segment(idx_h, w_per_tile, acc_row, is_pos, scale):
            n_chunks = w_per_tile // _W_CHUNK
            base_w = wid * w_per_tile
            n_idx = w_per_tile * _CTX
            seg_off = pl.multiple_of(base_w * _CTX, 8)
            pltpu.sync_copy(
                idx_h.at[pl.ds(seg_off, n_idx)], idx_all.at[pl.ds(0, n_idx)]
            )
            issue_gather(0, 0, gsems[0])

            @pl.loop(0, n_chunks, step=2)
            def chunk_loop(ci):
                for b in range(2):
                    cur = ci + b

                    @pl.when(cur + 1 < n_chunks)
                    def _prefetch():
                        issue_gather(cur + 1, 1 - b, gsems[1 - b])

                    wait_gather(b, gsems[b])

                    @pl.loop(0, _W_CHUNK // 8, unroll=2)
                    def blk_loop(blk):
                        group = []
                        for dw in range(8):
                            row = (blk * 8 + dw) * _CTX
                            s = [
                                rows_v[b, row, pl.ds(k * 16, 16)]
                                for k in range(8)
                            ]
                            for c in range(_PAIRS):
                                prods = [
                                    s[k] * rows_v[b, row + 1 + c, pl.ds(k * 16, 16)]
                                    for k in range(8)
                                ]
                                for stride in (4, 2, 1):
                                    prods = [
                                        prods[i] + prods[i + stride]
                                        for i in range(stride)
                                    ]
                                group.append(hsum(prods[0]))
                                if len(group) == 16:
                                    flush_group(group, acc_row, is_pos, scale)
                                    group = []

        run_segment(pos_h, pos_w_per_tile, 0, True, pos_scale)
        run_segment(neg_h, neg_w_per_tile, 1, False, neg_scale)

        out_off = pl.multiple_of(wid * 8, 8)
        pltpu.sync_copy(acc_v, out_h.at[pl.ds(out_off, 8)])

    return kern(emb, pos_idx, neg_idx)


@jax.jit
def kernel(emb, pos_rw, neg_rw):
    pos_b, ctx = pos_rw.shape
    neg_b, _ = neg_rw.shape
    assert ctx == _CTX and emb.shape[1] == _DIM
    parts = _sc_loss_partials(
        emb, pos_rw.reshape(-1), neg_rw.reshape(-1), pos_b, neg_b
    )
    parts = parts.reshape(_NW, 8, _DIM)
    return jnp.sum(parts[:, :2, :16])


# speedup vs baseline: 1.5622x; 1.5622x over previous
import functools

import jax
import jax.numpy as jnp
from jax import lax
from jax.experimental import pallas as pl
from jax.experimental.pallas import tpu as pltpu
from jax.experimental.pallas import tpu_sc as plsc

_NC = 2
_NS = 16
_NW = _NC * _NS

_DIM = 128
_CTX = 7
_PAIRS = _CTX - 1

_W_CHUNK = 32
_ROWS_CHUNK = _W_CHUNK * _CTX

_LOG1P_Q = (
    0.999999518605956,
    -0.49996356582566853,
    0.332652558440823,
    -0.24453388970156367,
    0.17659864767022154,
    -0.10679931279902478,
    0.04365928884096363,
    -0.008466410403222956,
)


def _sc_loss_partials(emb, pos_idx, neg_idx, pos_b, neg_b):
    pos_w_per_tile = pos_b // _NW
    neg_w_per_tile = neg_b // _NW
    pos_scale = 1.0 / float(pos_b * _PAIRS)
    neg_scale = 1.0 / float(neg_b * _PAIRS)
    mesh = plsc.VectorSubcoreMesh(core_axis_name="c", subcore_axis_name="s")

    idx_per_tile = (pos_w_per_tile + neg_w_per_tile) * _CTX

    @functools.partial(
        pl.kernel,
        mesh=mesh,
        out_type=jax.ShapeDtypeStruct((_NW * 8, _DIM), jnp.float32),
        scratch_types=[
            pltpu.VMEM((idx_per_tile,), jnp.int32),
            pltpu.VMEM((2, _ROWS_CHUNK, _DIM), jnp.float32),
            pltpu.VMEM((8, _DIM), jnp.float32),
            pltpu.SemaphoreType.DMA,
            pltpu.SemaphoreType.DMA,
        ],
    )
    def kern(emb_h, pos_h, neg_h, out_h, idx_all, rows_v, acc_v, gsem0, gsem1):
        wid = lax.axis_index("s") * _NC + lax.axis_index("c")
        gsems = [gsem0, gsem1]
        zeros = jnp.zeros((16,), jnp.float32)

        row_iota = lax.iota(jnp.int32, 16)
        dnums = lax.GatherDimensionNumbers(
            offset_dims=(), collapsed_slice_dims=(0,), start_index_map=(0,)
        )
        perms = [
            (row_iota ^ m)[:, None] for m in (8, 4, 2, 1)
        ]
        sel_masks = [(row_iota & bit) != 0 for bit in (1, 2, 4, 8)]

        def hsum(x):
            for perm in perms:
                x = x + lax.gather(
                    x, perm, dnums, (1,),
                    mode=lax.GatherScatterMode.PROMISE_IN_BOUNDS,
                )
            return x

        def issue_gather(ci, b, sem):
            off = pl.multiple_of(ci * _ROWS_CHUNK, 8)
            pltpu.async_copy(
                emb_h.at[idx_all.at[pl.ds(off, 128)]],
                rows_v.at[b].at[pl.ds(0, 128)],
                sem,
            )
            pltpu.async_copy(
                emb_h.at[idx_all.at[pl.ds(off + 128, 96)]],
                rows_v.at[b].at[pl.ds(128, 96)],
                sem,
            )

        def wait_gather(b, sem):
            pltpu.make_async_copy(
                emb_h.at[idx_all.at[pl.ds(0, 128)]],
                rows_v.at[b].at[pl.ds(0, 128)],
                sem,
            ).wait()
            pltpu.make_async_copy(
                emb_h.at[idx_all.at[pl.ds(0, 96)]],
                rows_v.at[b].at[pl.ds(128, 96)],
                sem,
            ).wait()

        def flush_group(group, sign_vec, scale_vec):
            cur = group
            for mask in sel_masks:
                cur = [
                    jnp.where(mask, cur[2 * j + 1], cur[2 * j])
                    for j in range(len(cur) // 2)
                ]
            x = cur[0]
            u = jnp.exp(-jnp.abs(x))
            c = [jnp.float32(v) for v in _LOG1P_Q]
            u2 = u * u
            u4 = u2 * u2
            q = (c[0] + c[1] * u + u2 * (c[2] + c[3] * u)) + u4 * (
                (c[4] + c[5] * u) + u2 * (c[6] + c[7] * u)
            )
            l1p = u * q
            contrib = l1p + jnp.maximum(x * sign_vec, 0.0)
            acc_v[0, pl.ds(0, 16)] = (
                acc_v[0, pl.ds(0, 16)] + contrib * scale_vec
            )

        n_pos_chunks = pos_w_per_tile // _W_CHUNK
        n_chunks = n_pos_chunks + neg_w_per_tile // _W_CHUNK
        pos_n_idx = pos_w_per_tile * _CTX
        neg_n_idx = neg_w_per_tile * _CTX
        acc_v[0, pl.ds(0, 16)] = zeros
        pltpu.sync_copy(
            pos_h.at[pl.ds(pl.multiple_of(wid * pos_n_idx, 8), pos_n_idx)],
            idx_all.at[pl.ds(0, pos_n_idx)],
        )
        pltpu.sync_copy(
            neg_h.at[pl.ds(pl.multiple_of(wid * neg_n_idx, 8), neg_n_idx)],
            idx_all.at[pl.ds(pos_n_idx, neg_n_idx)],
        )
        issue_gather(0, 0, gsems[0])

        @pl.loop(0, n_chunks, step=2)
        def chunk_loop(ci):
            for b in range(2):
                cur = ci + b

                @pl.when(cur + 1 < n_chunks)
                def _prefetch():
                    issue_gather(cur + 1, 1 - b, gsems[1 - b])

                wait_gather(b, gsems[b])

                is_pos = cur < n_pos_chunks
                sign_s = jnp.where(is_pos, jnp.float32(-1.0), jnp.float32(1.0))
                scale_s = jnp.where(
                    is_pos, jnp.float32(pos_scale), jnp.float32(neg_scale)
                )
                sign_vec = jnp.broadcast_to(sign_s, (16,))
                scale_vec = jnp.broadcast_to(scale_s, (16,))

                @pl.loop(0, _W_CHUNK // 8)
                def blk_loop(blk):
                    group = []
                    for dw in range(8):
                        row = (blk * 8 + dw) * _CTX
                        s = [
                            rows_v[b, row, pl.ds(k * 16, 16)]
                            for k in range(8)
                        ]
                        for c in range(_PAIRS):
                            prods = [
                                s[k] * rows_v[b, row + 1 + c, pl.ds(k * 16, 16)]
                                for k in range(8)
                            ]
                            for stride in (4, 2, 1):
                                prods = [
                                    prods[i] + prods[i + stride]
                                    for i in range(stride)
                                ]
                            group.append(hsum(prods[0]))
                            if len(group) == 16:
                                flush_group(group, sign_vec, scale_vec)
                                group = []

        out_off = pl.multiple_of(wid * 8, 8)
        pltpu.sync_copy(acc_v, out_h.at[pl.ds(out_off, 8)])

    return kern(emb, pos_idx, neg_idx)


@jax.jit
def kernel(emb, pos_rw, neg_rw):
    pos_b, ctx = pos_rw.shape
    neg_b, _ = neg_rw.shape
    assert ctx == _CTX and emb.shape[1] == _DIM
    parts = _sc_loss_partials(
        emb, pos_rw.reshape(-1), neg_rw.reshape(-1), pos_b, neg_b
    )
    parts = parts.reshape(_NW, 8, _DIM)
    return jnp.sum(parts[:, 0, :16])


# speedup vs baseline: 1.5699x; 1.0050x over previous
import functools

import jax
import jax.numpy as jnp
from jax import lax
from jax.experimental import pallas as pl
from jax.experimental.pallas import tpu as pltpu
from jax.experimental.pallas import tpu_sc as plsc

_NC = 2
_NS = 16
_NW = _NC * _NS

_DIM = 128
_CTX = 7
_PAIRS = _CTX - 1

_W_CHUNK = 32
_ROWS_CHUNK = _W_CHUNK * _CTX
_NBUF = 2

_LOG1P_Q = (
    0.999999518605956,
    -0.49996356582566853,
    0.332652558440823,
    -0.24453388970156367,
    0.17659864767022154,
    -0.10679931279902478,
    0.04365928884096363,
    -0.008466410403222956,
)


def _sc_loss_partials(emb, pos_idx, neg_idx, pos_b, neg_b):
    pos_w_per_tile = pos_b // _NW
    neg_w_per_tile = neg_b // _NW
    pos_scale = 1.0 / float(pos_b * _PAIRS)
    neg_scale = 1.0 / float(neg_b * _PAIRS)
    mesh = plsc.VectorSubcoreMesh(core_axis_name="c", subcore_axis_name="s")

    idx_per_tile = (pos_w_per_tile + neg_w_per_tile) * _CTX

    @functools.partial(
        pl.kernel,
        mesh=mesh,
        out_type=jax.ShapeDtypeStruct((_NW * 8, _DIM), jnp.float32),
        scratch_types=[
            pltpu.VMEM((idx_per_tile,), jnp.int32),
            pltpu.VMEM((_NBUF * _ROWS_CHUNK, _DIM), jnp.float32),
            pltpu.VMEM((8, _DIM), jnp.float32),
        ] + [pltpu.SemaphoreType.DMA] * _NBUF,
    )
    def kern(emb_h, pos_h, neg_h, out_h, idx_all, rows_v, acc_v, *gsems):
        wid = lax.axis_index("s") * _NC + lax.axis_index("c")
        zeros = jnp.zeros((16,), jnp.float32)

        row_iota = lax.iota(jnp.int32, 16)
        dnums = lax.GatherDimensionNumbers(
            offset_dims=(), collapsed_slice_dims=(0,), start_index_map=(0,)
        )
        perms = [
            (row_iota ^ m)[:, None] for m in (8, 4, 2, 1)
        ]
        sel_masks = [(row_iota & bit) != 0 for bit in (1, 2, 4, 8)]

        def hsum(x):
            for perm in perms:
                x = x + lax.gather(
                    x, perm, dnums, (1,),
                    mode=lax.GatherScatterMode.PROMISE_IN_BOUNDS,
                )
            return x

        def issue_gather(ci, b):
            off = pl.multiple_of(ci * _ROWS_CHUNK, 8)
            base = b * _ROWS_CHUNK
            pltpu.async_copy(
                emb_h.at[idx_all.at[pl.ds(off, 128)]],
                rows_v.at[pl.ds(base, 128)],
                gsems[b],
            )
            pltpu.async_copy(
                emb_h.at[idx_all.at[pl.ds(off + 128, 96)]],
                rows_v.at[pl.ds(base + 128, 96)],
                gsems[b],
            )

        def wait_gather(b):
            base = b * _ROWS_CHUNK
            pltpu.make_async_copy(
                emb_h.at[idx_all.at[pl.ds(0, 128)]],
                rows_v.at[pl.ds(base, 128)],
                gsems[b],
            ).wait()
            pltpu.make_async_copy(
                emb_h.at[idx_all.at[pl.ds(0, 96)]],
                rows_v.at[pl.ds(base + 128, 96)],
                gsems[b],
            ).wait()

        def flush_group(group, sign_vec, scale_vec):
            cur = group
            for mask in sel_masks:
                cur = [
                    jnp.where(mask, cur[2 * j + 1], cur[2 * j])
                    for j in range(len(cur) // 2)
                ]
            x = cur[0]
            u = jnp.exp(-jnp.abs(x))
            c = [jnp.float32(v) for v in _LOG1P_Q]
            u2 = u * u
            u4 = u2 * u2
            q = (c[0] + c[1] * u + u2 * (c[2] + c[3] * u)) + u4 * (
                (c[4] + c[5] * u) + u2 * (c[6] + c[7] * u)
            )
            l1p = u * q
            contrib = l1p + jnp.maximum(x * sign_vec, 0.0)
            acc_v[0, pl.ds(0, 16)] = (
                acc_v[0, pl.ds(0, 16)] + contrib * scale_vec
            )

        n_pos_chunks = pos_w_per_tile // _W_CHUNK
        n_chunks = n_pos_chunks + neg_w_per_tile // _W_CHUNK
        pos_n_idx = pos_w_per_tile * _CTX
        neg_n_idx = neg_w_per_tile * _CTX
        acc_v[0, pl.ds(0, 16)] = zeros
        pltpu.sync_copy(
            pos_h.at[pl.ds(pl.multiple_of(wid * pos_n_idx, 8), pos_n_idx)],
            idx_all.at[pl.ds(0, pos_n_idx)],
        )
        pltpu.sync_copy(
            neg_h.at[pl.ds(pl.multiple_of(wid * neg_n_idx, 8), neg_n_idx)],
            idx_all.at[pl.ds(pos_n_idx, neg_n_idx)],
        )
        for b in range(_NBUF - 1):
            issue_gather(b, b)

        @pl.loop(0, n_chunks)
        def chunk_loop(cur):
            par = lax.rem(cur, _NBUF)

            @pl.when(cur + _NBUF - 1 < n_chunks)
            def _prefetch():
                nxt = cur + _NBUF - 1
                npar = lax.rem(nxt, _NBUF)
                for b in range(_NBUF):
                    @pl.when(npar == b)
                    def _issue(b=b):
                        issue_gather(nxt, b)

            for b in range(_NBUF):
                @pl.when(par == b)
                def _wait(b=b):
                    wait_gather(b)

            is_pos = cur < n_pos_chunks
            sign_s = jnp.where(is_pos, jnp.float32(-1.0), jnp.float32(1.0))
            scale_s = jnp.where(
                is_pos, jnp.float32(pos_scale), jnp.float32(neg_scale)
            )
            sign_vec = jnp.broadcast_to(sign_s, (16,))
            scale_vec = jnp.broadcast_to(scale_s, (16,))
            buf_base = par * _ROWS_CHUNK

            @pl.loop(0, _W_CHUNK // 8)
            def blk_loop(blk):
                group = []
                for dw in range(8):
                    row = buf_base + (blk * 8 + dw) * _CTX
                    s = [
                        rows_v[row, pl.ds(k * 16, 16)]
                        for k in range(8)
                    ]
                    for c in range(_PAIRS):
                        prods = [
                            s[k] * rows_v[row + 1 + c, pl.ds(k * 16, 16)]
                            for k in range(8)
                        ]
                        for stride in (4, 2, 1):
                            prods = [
                                prods[i] + prods[i + stride]
                                for i in range(stride)
                            ]
                        group.append(hsum(prods[0]))
                        if len(group) == 16:
                            flush_group(group, sign_vec, scale_vec)
                            group = []

        out_off = pl.multiple_of(wid * 8, 8)
        pltpu.sync_copy(acc_v, out_h.at[pl.ds(out_off, 8)])

    return kern(emb, pos_idx, neg_idx)


@jax.jit
def kernel(emb, pos_rw, neg_rw):
    pos_b, ctx = pos_rw.shape
    neg_b, _ = neg_rw.shape
    assert ctx == _CTX and emb.shape[1] == _DIM
    parts = _sc_loss_partials(
        emb, pos_rw.reshape(-1), neg_rw.reshape(-1), pos_b, neg_b
    )
    parts = parts.reshape(_NW, 8, _DIM)
    return jnp.sum(parts[:, 0, :16])


# speedup vs baseline: 1.6817x; 1.0712x over previous
import functools

import jax
import jax.numpy as jnp
from jax import lax
from jax.experimental import pallas as pl
from jax.experimental.pallas import tpu as pltpu
from jax.experimental.pallas import tpu_sc as plsc

_NC = 2
_NS = 16
_NW = _NC * _NS

_DIM = 128
_CTX = 7
_PAIRS = _CTX - 1

_W_CHUNK = 32
_ROWS_CHUNK = _W_CHUNK * _CTX
_NBUF = 3

_LOG1P_Q = (
    0.999999518605956,
    -0.49996356582566853,
    0.332652558440823,
    -0.24453388970156367,
    0.17659864767022154,
    -0.10679931279902478,
    0.04365928884096363,
    -0.008466410403222956,
)


def _sc_loss_partials(emb, pos_idx, neg_idx, pos_b, neg_b):
    pos_w_per_tile = pos_b // _NW
    neg_w_per_tile = neg_b // _NW
    pos_scale = 1.0 / float(pos_b * _PAIRS)
    neg_scale = 1.0 / float(neg_b * _PAIRS)
    mesh = plsc.VectorSubcoreMesh(core_axis_name="c", subcore_axis_name="s")

    idx_per_tile = (pos_w_per_tile + neg_w_per_tile) * _CTX

    @functools.partial(
        pl.kernel,
        mesh=mesh,
        out_type=jax.ShapeDtypeStruct((_NW * 8, _DIM), jnp.float32),
        scratch_types=[
            pltpu.VMEM((idx_per_tile,), jnp.int32),
            pltpu.VMEM((_NBUF * _ROWS_CHUNK, _DIM), jnp.float32),
            pltpu.VMEM((8, _DIM), jnp.float32),
        ] + [pltpu.SemaphoreType.DMA] * _NBUF,
    )
    def kern(emb_h, pos_h, neg_h, out_h, idx_all, rows_v, acc_v, *gsems):
        wid = lax.axis_index("s") * _NC + lax.axis_index("c")
        zeros = jnp.zeros((16,), jnp.float32)

        row_iota = lax.iota(jnp.int32, 16)
        dnums = lax.GatherDimensionNumbers(
            offset_dims=(), collapsed_slice_dims=(0,), start_index_map=(0,)
        )
        perms = [
            (row_iota ^ m)[:, None] for m in (8, 4, 2, 1)
        ]
        sel_masks = [(row_iota & bit) != 0 for bit in (1, 2, 4, 8)]

        def hsum(x):
            for perm in perms:
                x = x + lax.gather(
                    x, perm, dnums, (1,),
                    mode=lax.GatherScatterMode.PROMISE_IN_BOUNDS,
                )
            return x

        def issue_gather(ci, b):
            off = pl.multiple_of(ci * _ROWS_CHUNK, 8)
            base = b * _ROWS_CHUNK
            pltpu.async_copy(
                emb_h.at[idx_all.at[pl.ds(off, 128)]],
                rows_v.at[pl.ds(base, 128)],
                gsems[b],
            )
            pltpu.async_copy(
                emb_h.at[idx_all.at[pl.ds(off + 128, 96)]],
                rows_v.at[pl.ds(base + 128, 96)],
                gsems[b],
            )

        def wait_gather(b):
            base = b * _ROWS_CHUNK
            pltpu.make_async_copy(
                emb_h.at[idx_all.at[pl.ds(0, 128)]],
                rows_v.at[pl.ds(base, 128)],
                gsems[b],
            ).wait()
            pltpu.make_async_copy(
                emb_h.at[idx_all.at[pl.ds(0, 96)]],
                rows_v.at[pl.ds(base + 128, 96)],
                gsems[b],
            ).wait()

        def flush_group(group, sign_vec, scale_vec):
            cur = group
            for mask in sel_masks:
                cur = [
                    jnp.where(mask, cur[2 * j + 1], cur[2 * j])
                    for j in range(len(cur) // 2)
                ]
            x = cur[0]
            u = jnp.exp(-jnp.abs(x))
            c = [jnp.float32(v) for v in _LOG1P_Q]
            u2 = u * u
            u4 = u2 * u2
            q = (c[0] + c[1] * u + u2 * (c[2] + c[3] * u)) + u4 * (
                (c[4] + c[5] * u) + u2 * (c[6] + c[7] * u)
            )
            l1p = u * q
            contrib = l1p + jnp.maximum(x * sign_vec, 0.0)
            acc_v[0, pl.ds(0, 16)] = (
                acc_v[0, pl.ds(0, 16)] + contrib * scale_vec
            )

        n_pos_chunks = pos_w_per_tile // _W_CHUNK
        n_chunks = n_pos_chunks + neg_w_per_tile // _W_CHUNK
        pos_n_idx = pos_w_per_tile * _CTX
        neg_n_idx = neg_w_per_tile * _CTX
        acc_v[0, pl.ds(0, 16)] = zeros
        pltpu.sync_copy(
            pos_h.at[pl.ds(pl.multiple_of(wid * pos_n_idx, 8), pos_n_idx)],
            idx_all.at[pl.ds(0, pos_n_idx)],
        )
        pltpu.sync_copy(
            neg_h.at[pl.ds(pl.multiple_of(wid * neg_n_idx, 8), neg_n_idx)],
            idx_all.at[pl.ds(pos_n_idx, neg_n_idx)],
        )
        for b in range(_NBUF - 1):
            issue_gather(b, b)

        @pl.loop(0, n_chunks)
        def chunk_loop(cur):
            par = lax.rem(cur, _NBUF)

            @pl.when(cur + _NBUF - 1 < n_chunks)
            def _prefetch():
                nxt = cur + _NBUF - 1
                npar = lax.rem(nxt, _NBUF)
                for b in range(_NBUF):
                    @pl.when(npar == b)
                    def _issue(b=b):
                        issue_gather(nxt, b)

            for b in range(_NBUF):
                @pl.when(par == b)
                def _wait(b=b):
                    wait_gather(b)

            is_pos = cur < n_pos_chunks
            sign_s = jnp.where(is_pos, jnp.float32(-1.0), jnp.float32(1.0))
            scale_s = jnp.where(
                is_pos, jnp.float32(pos_scale), jnp.float32(neg_scale)
            )
            sign_vec = jnp.broadcast_to(sign_s, (16,))
            scale_vec = jnp.broadcast_to(scale_s, (16,))
            buf_base = par * _ROWS_CHUNK

            @pl.loop(0, _W_CHUNK // 8)
            def blk_loop(blk):
                group = []
                for dw in range(8):
                    row = buf_base + (blk * 8 + dw) * _CTX
                    s = [
                        rows_v[row, pl.ds(k * 16, 16)]
                        for k in range(8)
                    ]
                    for c in range(_PAIRS):
                        prods = [
                            s[k] * rows_v[row + 1 + c, pl.ds(k * 16, 16)]
                            for k in range(8)
                        ]
                        for stride in (4, 2, 1):
                            prods = [
                                prods[i] + prods[i + stride]
                                for i in range(stride)
                            ]
                        group.append(hsum(prods[0]))
                        if len(group) == 16:
                            flush_group(group, sign_vec, scale_vec)
                            group = []

        out_off = pl.multiple_of(wid * 8, 8)
        pltpu.sync_copy(acc_v, out_h.at[pl.ds(out_off, 8)])

    return kern(emb, pos_idx, neg_idx)


@jax.jit
def kernel(emb, pos_rw, neg_rw):
    pos_b, ctx = pos_rw.shape
    neg_b, _ = neg_rw.shape
    assert ctx == _CTX and emb.shape[1] == _DIM
    parts = _sc_loss_partials(
        emb, pos_rw.reshape(-1), neg_rw.reshape(-1), pos_b, neg_b
    )
    parts = parts.reshape(_NW, 8, _DIM)
    return jnp.sum(parts[:, 0, :16])
